# baseline (speedup 1.0000x reference)
import functools

import jax
import jax.numpy as jnp
from jax import lax
from jax.experimental import pallas as pl
from jax.experimental.pallas import tpu as pltpu
from jax.experimental.pallas import tpu_sc as plsc

N = 10000
E = 320000
D = 128

NW = 32
CHUNK = 128
TOT_CHUNKS = 2528
C0 = 120
C1 = TOT_CHUNKS // 16 - C0
MAXC = max(C0, C1)
E_PAD = TOT_CHUNKS * CHUNK
N_PAD = 10240
ROWS_PS = N_PAD // 16


def _sc_segment_sum(h, src3, dst3, zeros):
    mesh = plsc.VectorSubcoreMesh(core_axis_name="c", subcore_axis_name="s")

    @functools.partial(
        pl.kernel,
        out_type=jax.ShapeDtypeStruct((2, N_PAD, D), jnp.float32),
        mesh=mesh,
        scratch_types=[
            pltpu.VMEM((MAXC, 1, CHUNK), jnp.int32),
            pltpu.VMEM((MAXC, 1, CHUNK), jnp.int32),
            pltpu.VMEM((CHUNK, D), jnp.float32),
            pltpu.VMEM_SHARED((N_PAD, D), jnp.float32),
            pltpu.SemaphoreType.DMA,
        ],
    )
    def k(h_hbm, src_hbm, dst_hbm, z_hbm, out_hbm, idx_s, idx_d, rows,
          acc, sem):
        c = lax.axis_index("c")
        s = lax.axis_index("s")
        start = lax.select(c == 0, s * C0, 16 * C0 + s * C1)
        nc = lax.select(c == 0, C0, C1)

        pltpu.sync_copy(src_hbm.at[pl.ds(start, MAXC)], idx_s)
        pltpu.sync_copy(dst_hbm.at[pl.ds(start, MAXC)], idx_d)

        pltpu.sync_copy(z_hbm, acc.at[pl.ds(s * ROWS_PS, ROWS_PS)])
        plsc.subcore_barrier()

        def step(j, carry):
            pltpu.async_copy(h_hbm.at[idx_s.at[j, 0]], rows, sem).wait()
            pltpu.sync_copy(rows, acc.at[idx_d.at[j, 0]], add=True)
            return carry

        lax.fori_loop(0, nc, step, 0)
        plsc.subcore_barrier()

        pltpu.sync_copy(
            acc.at[pl.ds(s * ROWS_PS, ROWS_PS)],
            out_hbm.at[c, pl.ds(s * ROWS_PS, ROWS_PS)],
        )

    return k(h, src3, dst3, zeros)


def _tc_dense(p, h, Wr, br2, Wo, relu):
    BLK = 400
    grid = (N // BLK,)

    def body(p0, p1, h_ref, wr, b, wo, o):
        agg = p0[0] + p1[0]
        acc = (
            jnp.dot(agg, wr[...], preferred_element_type=jnp.float32)
            + jnp.dot(h_ref[...], wo[...], preferred_element_type=jnp.float32)
            + b[...]
        )
        o[...] = jnp.maximum(acc, 0.0) if relu else acc

    return pl.pallas_call(
        body,
        grid=grid,
        in_specs=[
            pl.BlockSpec((1, BLK, D), lambda i: (0, i, 0)),
            pl.BlockSpec((1, BLK, D), lambda i: (1, i, 0)),
            pl.BlockSpec((BLK, D), lambda i: (i, 0)),
            pl.BlockSpec((D, D), lambda i: (0, 0)),
            pl.BlockSpec((1, D), lambda i: (0, 0)),
            pl.BlockSpec((D, D), lambda i: (0, 0)),
        ],
        out_specs=pl.BlockSpec((BLK, D), lambda i: (i, 0)),
        out_shape=jax.ShapeDtypeStruct((N, D), jnp.float32),
    )(p, p, h, Wr, br2, Wo)


def kernel(x, edge_index, W_rel0, b_rel0, W_root0, W_rel1, b_rel1, W_root1,
           W_rel2, b_rel2, W_root2):
    src = edge_index[0].astype(jnp.int32)
    dst = edge_index[1].astype(jnp.int32)
    pad_n = (TOT_CHUNKS + MAXC) * CHUNK - E
    src3 = jnp.pad(src, (0, pad_n)).reshape(-1, 1, CHUNK)
    pad_dst = N + (jnp.arange(pad_n, dtype=jnp.int32) % (N_PAD - N))
    dst3 = jnp.concatenate([dst, pad_dst]).reshape(-1, 1, CHUNK)
    zeros = jnp.zeros((ROWS_PS, D), jnp.float32)

    layers = [
        (W_rel0, b_rel0, W_root0, True),
        (W_rel1, b_rel1, W_root1, True),
        (W_rel2, b_rel2, W_root2, False),
    ]
    h = x
    for Wr, br, Wo, relu in layers:
        p = _sc_segment_sum(h, src3, dst3, zeros)
        h = _tc_dense(p, h, Wr, br.reshape(1, D), Wo, relu)
    return h


# speedup vs baseline: 1.0151x; 1.0151x over previous
import functools

import jax
import jax.numpy as jnp
from jax import lax
from jax.experimental import pallas as pl
from jax.experimental.pallas import tpu as pltpu
from jax.experimental.pallas import tpu_sc as plsc

N = 10000
E = 320000
D = 128

NW = 32
CHUNK = 128
TOT_CHUNKS = 2528
C0 = 116
C1 = TOT_CHUNKS // 16 - C0
MAXC = max(C0, C1)
E_PAD = TOT_CHUNKS * CHUNK
N_PAD = 10240
ROWS_PS = N_PAD // 16


def _sc_segment_sum(h, src3, dst3, zeros):
    mesh = plsc.VectorSubcoreMesh(core_axis_name="c", subcore_axis_name="s")

    @functools.partial(
        pl.kernel,
        out_type=jax.ShapeDtypeStruct((2, N_PAD, D), jnp.float32),
        mesh=mesh,
        scratch_types=[
            pltpu.VMEM((MAXC, 1, CHUNK), jnp.int32),
            pltpu.VMEM((MAXC, 1, CHUNK), jnp.int32),
            pltpu.VMEM((CHUNK, D), jnp.float32),
            pltpu.VMEM_SHARED((N_PAD, D), jnp.float32),
            pltpu.SemaphoreType.DMA,
        ],
    )
    def k(h_hbm, src_hbm, dst_hbm, z_hbm, out_hbm, idx_s, idx_d, rows,
          acc, sem):
        c = lax.axis_index("c")
        s = lax.axis_index("s")
        start = lax.select(c == 0, s * C0, 16 * C0 + s * C1)
        nc = lax.select(c == 0, C0, C1)

        pltpu.sync_copy(src_hbm.at[pl.ds(start, MAXC)], idx_s)
        pltpu.sync_copy(dst_hbm.at[pl.ds(start, MAXC)], idx_d)

        pltpu.sync_copy(z_hbm, acc.at[pl.ds(s * ROWS_PS, ROWS_PS)])
        plsc.subcore_barrier()

        def step(j, carry):
            pltpu.async_copy(h_hbm.at[idx_s.at[j, 0]], rows, sem).wait()
            pltpu.sync_copy(rows, acc.at[idx_d.at[j, 0]], add=True)
            return carry

        lax.fori_loop(0, nc, step, 0)
        plsc.subcore_barrier()

        pltpu.sync_copy(
            acc.at[pl.ds(s * ROWS_PS, ROWS_PS)],
            out_hbm.at[c, pl.ds(s * ROWS_PS, ROWS_PS)],
        )

    return k(h, src3, dst3, zeros)


def _tc_dense(p, h, Wr, br2, Wo, relu):
    BLK = 400
    grid = (N // BLK,)

    def body(p0, p1, h_ref, wr, b, wo, o):
        agg = p0[0] + p1[0]
        acc = (
            jnp.dot(agg, wr[...], preferred_element_type=jnp.float32)
            + jnp.dot(h_ref[...], wo[...], preferred_element_type=jnp.float32)
            + b[...]
        )
        o[...] = jnp.maximum(acc, 0.0) if relu else acc

    return pl.pallas_call(
        body,
        grid=grid,
        in_specs=[
            pl.BlockSpec((1, BLK, D), lambda i: (0, i, 0)),
            pl.BlockSpec((1, BLK, D), lambda i: (1, i, 0)),
            pl.BlockSpec((BLK, D), lambda i: (i, 0)),
            pl.BlockSpec((D, D), lambda i: (0, 0)),
            pl.BlockSpec((1, D), lambda i: (0, 0)),
            pl.BlockSpec((D, D), lambda i: (0, 0)),
        ],
        out_specs=pl.BlockSpec((BLK, D), lambda i: (i, 0)),
        out_shape=jax.ShapeDtypeStruct((N, D), jnp.float32),
    )(p, p, h, Wr, br2, Wo)


def kernel(x, edge_index, W_rel0, b_rel0, W_root0, W_rel1, b_rel1, W_root1,
           W_rel2, b_rel2, W_root2):
    src = edge_index[0].astype(jnp.int32)
    dst = edge_index[1].astype(jnp.int32)
    pad_n = (TOT_CHUNKS + MAXC) * CHUNK - E
    src3 = jnp.pad(src, (0, pad_n)).reshape(-1, 1, CHUNK)
    pad_dst = N + (jnp.arange(pad_n, dtype=jnp.int32) % (N_PAD - N))
    dst3 = jnp.concatenate([dst, pad_dst]).reshape(-1, 1, CHUNK)
    zeros = jnp.zeros((ROWS_PS, D), jnp.float32)

    layers = [
        (W_rel0, b_rel0, W_root0, True),
        (W_rel1, b_rel1, W_root1, True),
        (W_rel2, b_rel2, W_root2, False),
    ]
    h = x
    for Wr, br, Wo, relu in layers:
        p = _sc_segment_sum(h, src3, dst3, zeros)
        h = _tc_dense(p, h, Wr, br.reshape(1, D), Wo, relu)
    return h
